# baseline (speedup 1.0000x reference)
import functools

import jax
import jax.numpy as jnp
from jax import lax
from jax.experimental import pallas as pl
from jax.experimental.pallas import tpu as pltpu
from jax.experimental.pallas import tpu_sc as plsc

N = 10000
E = 320000
D = 128
NPAD = 10240
NC = 2
NS = 16
NW = NC * NS
C = 64
NCHUNK = E // C
CHUNKS_PER_W = NCHUNK // NW
REMAINDER = NCHUNK - CHUNKS_PER_W * NW
NITER = CHUNKS_PER_W // 4
ROWS_PER_S = NPAD // NS

_sc_mesh = plsc.VectorSubcoreMesh(core_axis_name="c", subcore_axis_name="s")


@functools.partial(
    pl.kernel,
    out_type=(
        jax.ShapeDtypeStruct((NC, NPAD, D), jnp.float32),
        jax.ShapeDtypeStruct((NC, NPAD, 16), jnp.float32),
    ),
    mesh=_sc_mesh,
    scratch_types=[
        [pltpu.VMEM((2, C), jnp.int32) for _ in range(4)],
        [pltpu.VMEM((C, D), jnp.float32) for _ in range(4)],
        pltpu.VMEM((C, 16), jnp.float32),
        pltpu.VMEM_SHARED((NPAD, D), jnp.float32),
        pltpu.VMEM_SHARED((NPAD, 16), jnp.float32),
        [pltpu.SemaphoreType.DMA for _ in range(4)],
        [pltpu.SemaphoreType.DMA for _ in range(4)],
        [pltpu.SemaphoreType.DMA for _ in range(4)],
        [pltpu.SemaphoreType.DMA for _ in range(4)],
    ],
    compiler_params=pltpu.CompilerParams(use_tc_tiling_on_sc=False),
)
def _sc_agg(x_hbm, idx_hbm, zrows_hbm, zdeg_hbm, aggf_hbm, dego_hbm,
            idxb, rows, ones, aggsh, degsh, semi, semg, sems, semd):
    cid = lax.axis_index("c")
    sid = lax.axis_index("s")
    wid = sid * NC + cid
    base = wid * CHUNKS_PER_W

    for j in range(C):
        ones[j, :] = jnp.ones((16,), jnp.float32)

    def start_idx(ci, m):
        pltpu.async_copy(idx_hbm.at[ci], idxb[m], semi[m])

    def wait_idx(m):
        pltpu.make_async_copy(idx_hbm.at[0], idxb[m], semi[m]).wait()

    def start_gather(m, j):
        pltpu.async_copy(x_hbm.at[idxb[m].at[0]], rows[j], semg[j])

    def wait_gather(j):
        pltpu.make_async_copy(x_hbm.at[idxb[0].at[0]], rows[j], semg[j]).wait()

    def start_scatter(j, m):
        pltpu.async_copy(rows[j], aggsh.at[idxb[m].at[1]], sems[j], add=True)
        pltpu.async_copy(ones, degsh.at[idxb[m].at[1]], semd[j], add=True)

    def wait_scatter(j):
        pltpu.make_async_copy(rows[j], aggsh.at[idxb[0].at[1]], sems[j]).wait()
        pltpu.make_async_copy(ones, degsh.at[idxb[0].at[1]], semd[j]).wait()

    start_idx(base, 0)
    start_idx(base + 1, 1)
    wait_idx(0)
    start_gather(0, 0)
    wait_idx(1)
    start_gather(1, 1)

    pltpu.sync_copy(zrows_hbm, aggsh.at[pl.ds(sid * ROWS_PER_S, ROWS_PER_S)])
    pltpu.sync_copy(zdeg_hbm, degsh.at[pl.ds(sid * ROWS_PER_S, ROWS_PER_S)])
    plsc.subcore_barrier()

    def _loop_body(k, _):
        for j in range(4):
            cof = 4 * k + j
            jn = (j + 2) % 4
            wait_gather(j)
            start_scatter(j, j)

            @pl.when(cof >= 2)
            def _():
                wait_scatter(jn)

            @pl.when(cof + 2 < CHUNKS_PER_W)
            def _():
                start_idx(base + cof + 2, jn)
                wait_idx(jn)
                start_gather(jn, jn)

        return 0

    lax.fori_loop(0, NITER, _loop_body, 0)

    wait_scatter(2)
    wait_scatter(3)

    @pl.when(wid < REMAINDER)
    def _():
        ci = NW * CHUNKS_PER_W + wid
        pltpu.sync_copy(idx_hbm.at[ci], idxb[0])
        pltpu.async_copy(x_hbm.at[idxb[0].at[0]], rows[0], semg[0]).wait()
        pltpu.sync_copy(rows[0], aggsh.at[idxb[0].at[1]], add=True)
        pltpu.sync_copy(ones, degsh.at[idxb[0].at[1]], add=True)

    plsc.subcore_barrier()

    pltpu.sync_copy(aggsh.at[pl.ds(sid * ROWS_PER_S, ROWS_PER_S)],
                    aggf_hbm.at[cid, pl.ds(sid * ROWS_PER_S, ROWS_PER_S)])
    pltpu.sync_copy(degsh.at[pl.ds(sid * ROWS_PER_S, ROWS_PER_S)],
                    dego_hbm.at[cid, pl.ds(sid * ROWS_PER_S, ROWS_PER_S)])


_TC_R = 1024


def _tc_body(ag_ref, deg_ref, x_ref, w_ref, ws_ref, b_ref, o_ref):
    feat = ag_ref[0] + ag_ref[1]
    deg = deg_ref[0, :, 0:1] + deg_ref[1, :, 0:1]
    m = feat / jnp.maximum(deg, 1.0)
    o_ref[...] = jnp.maximum(
        jnp.dot(m, w_ref[...], preferred_element_type=jnp.float32)
        + jnp.dot(x_ref[...], ws_ref[...], preferred_element_type=jnp.float32)
        + b_ref[...],
        0.0,
    )


def _tc_finish(agg, deg, x, w, ws, b2):
    return pl.pallas_call(
        _tc_body,
        grid=(NPAD // _TC_R,),
        in_specs=[
            pl.BlockSpec((NC, _TC_R, D), lambda i: (0, i, 0)),
            pl.BlockSpec((NC, _TC_R, 16), lambda i: (0, i, 0)),
            pl.BlockSpec((_TC_R, D), lambda i: (i, 0)),
            pl.BlockSpec((D, D), lambda i: (0, 0)),
            pl.BlockSpec((D, D), lambda i: (0, 0)),
            pl.BlockSpec((1, D), lambda i: (0, 0)),
        ],
        out_specs=pl.BlockSpec((_TC_R, D), lambda i: (i, 0)),
        out_shape=jax.ShapeDtypeStruct((N, D), jnp.float32),
    )(agg, deg, x, w, ws, b2)


def kernel(x, edge_index, W, W_self, b):
    idx = edge_index.reshape(2, NCHUNK, C).transpose(1, 0, 2)
    zrows = jnp.zeros((ROWS_PER_S, D), jnp.float32)
    zdeg = jnp.zeros((ROWS_PER_S, 16), jnp.float32)
    aggf, dego = _sc_agg(x, idx, zrows, zdeg)
    return _tc_finish(aggf, dego, x, W, W_self, b.reshape(1, D))


# speedup vs baseline: 1.2028x; 1.2028x over previous
import functools

import jax
import jax.numpy as jnp
from jax import lax
from jax.experimental import pallas as pl
from jax.experimental.pallas import tpu as pltpu
from jax.experimental.pallas import tpu_sc as plsc

N = 10000
E = 320000
D = 128
NPAD = 10240
NC = 2
NS = 16
NW = NC * NS
C = 128
NCHUNK = E // C
CHUNKS_PER_W = NCHUNK // NW
REMAINDER = NCHUNK - CHUNKS_PER_W * NW
NPAIR = CHUNKS_PER_W // 2
NITER = (NPAIR - 1) // 2
ROWS_PER_S = NPAD // NS

_sc_mesh = plsc.VectorSubcoreMesh(core_axis_name="c", subcore_axis_name="s")


@functools.partial(
    pl.kernel,
    out_type=(
        jax.ShapeDtypeStruct((NC, NPAD, D), jnp.float32),
        jax.ShapeDtypeStruct((NC, NPAD, 16), jnp.float32),
    ),
    mesh=_sc_mesh,
    scratch_types=[
        pltpu.VMEM((2, 2, C), jnp.int32),
        pltpu.VMEM((2, 2, C), jnp.int32),
        pltpu.VMEM((2, C), jnp.int32),
        pltpu.VMEM((2, C), jnp.int32),
        pltpu.VMEM((C, D), jnp.float32),
        pltpu.VMEM((C, D), jnp.float32),
        pltpu.VMEM((C, 16), jnp.float32),
        pltpu.VMEM_SHARED((NPAD, D), jnp.float32),
        pltpu.VMEM_SHARED((NPAD, 16), jnp.float32),
        pltpu.SemaphoreType.DMA,
        pltpu.SemaphoreType.DMA,
        pltpu.SemaphoreType.DMA,
        pltpu.SemaphoreType.DMA,
        pltpu.SemaphoreType.DMA,
        pltpu.SemaphoreType.DMA,
        pltpu.SemaphoreType.DMA,
        pltpu.SemaphoreType.DMA,
    ],
    compiler_params=pltpu.CompilerParams(use_tc_tiling_on_sc=False),
)
def _sc_agg(x_hbm, idx_hbm, zrows_hbm, zdeg_hbm, aggf_hbm, dego_hbm,
            idxp0, idxp1, dpair0, dpair1, rows0, rows1, ones, aggsh, degsh,
            semi0, semi1, semg0, semg1, sems0, sems1, semd0, semd1):
    cid = lax.axis_index("c")
    sid = lax.axis_index("s")
    wid = sid * NC + cid
    base = wid * CHUNKS_PER_W

    for j in range(C):
        ones[j, :] = jnp.ones((16,), jnp.float32)

    def start_idx_pair(c0, bank, sem):
        pltpu.async_copy(idx_hbm.at[pl.ds(c0, 2)], bank, sem)

    def wait_idx(bank, sem):
        pltpu.make_async_copy(idx_hbm.at[pl.ds(0, 2)], bank, sem).wait()

    def start_gather(srcrow, buf, sem):
        pltpu.async_copy(x_hbm.at[srcrow], buf, sem)

    def wait_gather(buf, sem):
        pltpu.make_async_copy(x_hbm.at[idxp0.at[0, 0]], buf, sem).wait()

    def start_scatter(buf, dstrow, sem):
        pltpu.async_copy(buf, aggsh.at[dstrow], sem, add=True)

    def wait_scatter(buf, sem):
        pltpu.make_async_copy(buf, aggsh.at[idxp0.at[0, 1]], sem).wait()

    def start_deg(dstrow, sem):
        pltpu.async_copy(ones, degsh.at[dstrow], sem, add=True)

    def wait_deg2(dpair, sem):
        pltpu.make_async_copy(ones, degsh.at[dpair.at[0]], sem).wait()
        pltpu.make_async_copy(ones, degsh.at[dpair.at[0]], sem).wait()

    def stage(dpair, slot, bank, half):
        for t in range(C // 16):
            dpair[slot, pl.ds(16 * t, 16)] = bank[half, 1, pl.ds(16 * t, 16)]

    start_idx_pair(base, idxp0, semi0)
    start_idx_pair(base + 2, idxp1, semi1)
    wait_idx(idxp0, semi0)
    start_gather(idxp0.at[0, 0], rows0, semg0)

    pltpu.sync_copy(zrows_hbm, aggsh.at[pl.ds(sid * ROWS_PER_S, ROWS_PER_S)])
    pltpu.sync_copy(zdeg_hbm, degsh.at[pl.ds(sid * ROWS_PER_S, ROWS_PER_S)])
    plsc.subcore_barrier()

    def half_iter(k, c0, bank, banksem, dpair, dsem, refill_c, refill_bank,
                  refill_sem, do_refill, next_bank, next_banksem, wait_dsem):
        wait_gather(rows0, semg0)
        start_scatter(rows0, bank.at[0, 1], sems0)

        @pl.when(wait_dsem)
        def _():
            wait_deg2(dpair, dsem)
        stage(dpair, 0, bank, 0)
        start_deg(dpair.at[0], dsem)
        start_gather(bank.at[1, 0], rows1, semg1)
        wait_scatter(rows0, sems0)
        stage(dpair, 1, bank, 1)
        start_deg(dpair.at[1], dsem)
        wait_gather(rows1, semg1)
        @pl.when(do_refill)
        def _():
            start_idx_pair(refill_c, refill_bank, refill_sem)
        start_scatter(rows1, dpair.at[1], sems1)
        wait_idx(next_bank, next_banksem)
        start_gather(next_bank.at[0, 0], rows0, semg0)
        wait_scatter(rows1, sems1)

    def _loop_body(k, _):
        c0 = base + 4 * k
        half_iter(k, c0, idxp0, semi0, dpair0, semd0,
                  c0 + 4, idxp0, semi0, True, idxp1, semi1, k > 0)
        half_iter(k, c0 + 2, idxp1, semi1, dpair1, semd1,
                  c0 + 6, idxp1, semi1, k < NITER - 1, idxp0, semi0, k > 0)
        return 0

    lax.fori_loop(0, NITER, _loop_body, 0)

    wait_gather(rows0, semg0)
    start_scatter(rows0, idxp0.at[0, 1], sems0)
    wait_deg2(dpair0, semd0)
    stage(dpair0, 0, idxp0, 0)
    start_deg(dpair0.at[0], semd0)
    start_gather(idxp0.at[1, 0], rows1, semg1)
    wait_scatter(rows0, sems0)
    stage(dpair0, 1, idxp0, 1)
    start_deg(dpair0.at[1], semd0)
    wait_gather(rows1, semg1)
    start_scatter(rows1, dpair0.at[1], sems1)
    wait_scatter(rows1, sems1)
    wait_deg2(dpair0, semd0)
    wait_deg2(dpair1, semd1)

    @pl.when(wid < REMAINDER)
    def _():
        ci = NW * CHUNKS_PER_W + wid
        pltpu.sync_copy(idx_hbm.at[pl.ds(ci, 2)], idxp0)
        pltpu.async_copy(x_hbm.at[idxp0.at[0, 0]], rows0, semg0).wait()
        pltpu.sync_copy(rows0, aggsh.at[idxp0.at[0, 1]], add=True)
        pltpu.sync_copy(ones, degsh.at[idxp0.at[0, 1]], add=True)

    plsc.subcore_barrier()

    pltpu.sync_copy(aggsh.at[pl.ds(sid * ROWS_PER_S, ROWS_PER_S)],
                    aggf_hbm.at[cid, pl.ds(sid * ROWS_PER_S, ROWS_PER_S)])
    pltpu.sync_copy(degsh.at[pl.ds(sid * ROWS_PER_S, ROWS_PER_S)],
                    dego_hbm.at[cid, pl.ds(sid * ROWS_PER_S, ROWS_PER_S)])


_TC_R = 1024


def _tc_body(ag_ref, deg_ref, x_ref, w_ref, ws_ref, b_ref, o_ref):
    feat = ag_ref[0] + ag_ref[1]
    deg = deg_ref[0, :, 0:1] + deg_ref[1, :, 0:1]
    m = feat / jnp.maximum(deg, 1.0)
    o_ref[...] = jnp.maximum(
        jnp.dot(m, w_ref[...], preferred_element_type=jnp.float32)
        + jnp.dot(x_ref[...], ws_ref[...], preferred_element_type=jnp.float32)
        + b_ref[...],
        0.0,
    )


def _tc_finish(agg, deg, x, w, ws, b2):
    return pl.pallas_call(
        _tc_body,
        grid=(NPAD // _TC_R,),
        in_specs=[
            pl.BlockSpec((NC, _TC_R, D), lambda i: (0, i, 0)),
            pl.BlockSpec((NC, _TC_R, 16), lambda i: (0, i, 0)),
            pl.BlockSpec((_TC_R, D), lambda i: (i, 0)),
            pl.BlockSpec((D, D), lambda i: (0, 0)),
            pl.BlockSpec((D, D), lambda i: (0, 0)),
            pl.BlockSpec((1, D), lambda i: (0, 0)),
        ],
        out_specs=pl.BlockSpec((_TC_R, D), lambda i: (i, 0)),
        out_shape=jax.ShapeDtypeStruct((N, D), jnp.float32),
    )(agg, deg, x, w, ws, b2)


def kernel(x, edge_index, W, W_self, b):
    idx = edge_index.reshape(2, NCHUNK, C).transpose(1, 0, 2)
    zrows = jnp.zeros((ROWS_PER_S, D), jnp.float32)
    zdeg = jnp.zeros((ROWS_PER_S, 16), jnp.float32)
    aggf, dego = _sc_agg(x, idx, zrows, zdeg)
    return _tc_finish(aggf, dego, x, W, W_self, b.reshape(1, D))
